# baseline (speedup 1.0000x reference)
import functools

import jax
import jax.numpy as jnp
from jax import lax
from jax.experimental import pallas as pl
from jax.experimental.pallas import tpu as pltpu
from jax.experimental.pallas import tpu_sc as plsc

NC = 2
NS = 16
NW = NC * NS
LANES = 16
CHUNK = 128


def kernel(user, pos_item, neg_item, comment_tensor,
           user_embed_w, item_embed_w, text_embed_w):
    B = user.shape[0]
    Lw = comment_tensor.shape[1]
    D = user_embed_w.shape[1]
    NU = user_embed_w.shape[0]
    NI = item_embed_w.shape[0]
    bpw = B // NW
    nch = bpw // CHUNK

    uidx = user.astype(jnp.int32).reshape(NW, nch, CHUNK)
    pidx = pos_item.astype(jnp.int32).reshape(NW, nch, CHUNK)
    nidx = neg_item.astype(jnp.int32).reshape(NW, nch, CHUNK)
    cidx = comment_tensor.T.astype(jnp.int32).reshape(Lw, NW, nch, CHUNK)

    utp = jnp.pad(user_embed_w.T, ((0, 0), (0, (-NU) % 128)))
    itp = jnp.pad(item_embed_w.T, ((0, 0), (0, (-NI) % 128)))

    mesh = plsc.VectorSubcoreMesh(core_axis_name="c", subcore_axis_name="s")

    @functools.partial(
        pl.kernel,
        out_type=(jax.ShapeDtypeStruct((B,), jnp.float32),
                  jax.ShapeDtypeStruct((B,), jnp.float32)),
        mesh=mesh,
        compiler_params=pltpu.CompilerParams(needs_layout_passes=False,
                                             use_tc_tiling_on_sc=False),
        scratch_types=[
            pltpu.VMEM((nch, CHUNK), jnp.int32),
            pltpu.VMEM((nch, CHUNK), jnp.int32),
            pltpu.VMEM((nch, CHUNK), jnp.int32),
            pltpu.VMEM((Lw, nch, CHUNK), jnp.int32),
            pltpu.VMEM((D, bpw), jnp.float32),
            pltpu.VMEM((D, bpw), jnp.float32),
            pltpu.VMEM((D, bpw), jnp.float32),
            pltpu.VMEM((bpw, D), jnp.float32),
            pltpu.VMEM((bpw,), jnp.float32),
            pltpu.VMEM((bpw,), jnp.float32),
            pltpu.SemaphoreType.DMA,
            pltpu.SemaphoreType.DMA,
        ],
    )
    def run(uidx_h, pidx_h, nidx_h, cidx_h, utp_h, itp_h, tw_h,
            spos_h, sneg_h,
            uidx_v, pidx_v, nidx_v, cidx_v, uT_v, ipT_v, inT_v, c_v,
            sp_v, sn_v, sem_rows, sem_c):
        w = lax.axis_index("s") * NC + lax.axis_index("c")

        pltpu.sync_copy(uidx_h.at[w], uidx_v)
        pltpu.sync_copy(pidx_h.at[w], pidx_v)
        pltpu.sync_copy(nidx_h.at[w], nidx_v)
        pltpu.sync_copy(cidx_h.at[:, w], cidx_v)

        first = []
        for ch in range(nch):
            first.append(pltpu.async_copy(
                tw_h.at[cidx_v.at[0, ch]],
                c_v.at[pl.ds(ch * CHUNK, CHUNK)], sem_c))

        def fire_cols(d, carry):
            for ch in range(nch):
                sl = pl.ds(ch * CHUNK, CHUNK)
                pltpu.async_copy(utp_h.at[d].at[uidx_v.at[ch]],
                                 uT_v.at[d, sl], sem_rows)
                pltpu.async_copy(itp_h.at[d].at[pidx_v.at[ch]],
                                 ipT_v.at[d, sl], sem_rows)
                pltpu.async_copy(itp_h.at[d].at[nidx_v.at[ch]],
                                 inT_v.at[d, sl], sem_rows)
            return carry
        lax.fori_loop(0, D, fire_cols, 0)

        for dsc in first:
            dsc.wait()

        def fire(l, carry):
            for ch in range(nch):
                pltpu.async_copy(tw_h.at[cidx_v.at[l, ch]],
                                 c_v.at[pl.ds(ch * CHUNK, CHUNK)],
                                 sem_c, add=True)
            return carry
        lax.fori_loop(1, Lw, fire, 0)

        def drain_c(l, carry):
            for ch in range(nch):
                pltpu.make_async_copy(tw_h.at[cidx_v.at[l, ch]],
                                      c_v.at[pl.ds(ch * CHUNK, CHUNK)],
                                      sem_c).wait()
            return carry
        lax.fori_loop(1, Lw, drain_c, 0)

        def drain_cols(d, carry):
            for ch in range(nch):
                sl = pl.ds(ch * CHUNK, CHUNK)
                pltpu.make_async_copy(utp_h.at[d].at[uidx_v.at[ch]],
                                      uT_v.at[d, sl], sem_rows).wait()
                pltpu.make_async_copy(itp_h.at[d].at[pidx_v.at[ch]],
                                      ipT_v.at[d, sl], sem_rows).wait()
                pltpu.make_async_copy(itp_h.at[d].at[nidx_v.at[ch]],
                                      inT_v.at[d, sl], sem_rows).wait()
            return carry
        lax.fori_loop(0, D, drain_cols, 0)

        inv_l = jnp.float32(1.0 / Lw)
        iot = lax.iota(jnp.int32, LANES)

        def group(g, carry):
            r = g * LANES + iot
            sl = pl.ds(g * LANES, LANES)

            def dot_step(d, acc):
                s_p, s_n, s_c = acc
                dv = jnp.full((LANES,), d, jnp.int32)
                up = uT_v[d, sl]
                s_p = s_p + up * ipT_v[d, sl]
                s_n = s_n + up * inT_v[d, sl]
                s_c = s_c + up * plsc.load_gather(c_v, [r, dv])
                return (s_p, s_n, s_c)

            z = jnp.zeros((LANES,), jnp.float32)
            s_p, s_n, s_c = lax.fori_loop(0, D, dot_step, (z, z, z))
            sc = s_c * inv_l
            sp_v[sl] = s_p + sc
            sn_v[sl] = s_n + sc
            return carry
        lax.fori_loop(0, bpw // LANES, group, 0)

        base = w * bpw
        pltpu.sync_copy(sp_v, spos_h.at[pl.ds(base, bpw)])
        pltpu.sync_copy(sn_v, sneg_h.at[pl.ds(base, bpw)])

    sp, sn = run(uidx, pidx, nidx, cidx, utp, itp, text_embed_w)
    return sp, sn


# speedup vs baseline: 4.0924x; 4.0924x over previous
import functools

import jax
import jax.numpy as jnp
from jax import lax
from jax.experimental import pallas as pl
from jax.experimental.pallas import tpu as pltpu
from jax.experimental.pallas import tpu_sc as plsc

NC = 2
NS = 16
NW = NC * NS
LANES = 16
CHUNK = 128

_PARAMS = pltpu.CompilerParams(needs_layout_passes=False,
                               use_tc_tiling_on_sc=False)


def kernel(user, pos_item, neg_item, comment_tensor,
           user_embed_w, item_embed_w, text_embed_w):
    B = user.shape[0]
    Lw = comment_tensor.shape[1]
    D = user_embed_w.shape[1]
    bpw = B // NW
    nch = bpw // CHUNK
    ngr = bpw // LANES

    uidx = user.astype(jnp.int32).reshape(NW, nch, CHUNK)
    pidx = pos_item.astype(jnp.int32).reshape(NW, nch, CHUNK)
    nidx = neg_item.astype(jnp.int32).reshape(NW, nch, CHUNK)
    cidx = comment_tensor.T.astype(jnp.int32).reshape(Lw, NW, nch, CHUNK)

    mesh = plsc.VectorSubcoreMesh(core_axis_name="c", subcore_axis_name="s")

    @functools.partial(
        pl.kernel,
        out_type=(jax.ShapeDtypeStruct((B, D), jnp.float32),
                  jax.ShapeDtypeStruct((B,), jnp.float32)),
        mesh=mesh,
        compiler_params=_PARAMS,
        scratch_types=[
            pltpu.VMEM((nch, CHUNK), jnp.int32),
            pltpu.VMEM((Lw, nch, CHUNK), jnp.int32),
            pltpu.VMEM((bpw, D), jnp.float32),
            pltpu.VMEM((bpw, D), jnp.float32),
            pltpu.VMEM((bpw,), jnp.float32),
            pltpu.SemaphoreType.DMA,
            pltpu.SemaphoreType.DMA,
        ],
    )
    def run_a(uidx_h, cidx_h, uw_h, tw_h, ustage_h, cu_h,
              uidx_v, cidx_v, u_v, c_v, cu_v, sem_rows, sem_c):
        w = lax.axis_index("s") * NC + lax.axis_index("c")
        iot = lax.iota(jnp.int32, LANES)

        pltpu.sync_copy(uidx_h.at[w], uidx_v)
        pltpu.sync_copy(cidx_h.at[:, w], cidx_v)

        first = []
        for ch in range(nch):
            first.append(pltpu.async_copy(
                tw_h.at[cidx_v.at[0, ch]],
                c_v.at[pl.ds(ch * CHUNK, CHUNK)], sem_c))
        rows = []
        for ch in range(nch):
            rows.append(pltpu.async_copy(
                uw_h.at[uidx_v.at[ch]],
                u_v.at[pl.ds(ch * CHUNK, CHUNK)], sem_rows))
        for dsc in first:
            dsc.wait()

        def fire(l, carry):
            for ch in range(nch):
                pltpu.async_copy(tw_h.at[cidx_v.at[l, ch]],
                                 c_v.at[pl.ds(ch * CHUNK, CHUNK)],
                                 sem_c, add=True)
            return carry
        lax.fori_loop(1, Lw, fire, 0)

        def drain(l, carry):
            for ch in range(nch):
                pltpu.make_async_copy(tw_h.at[cidx_v.at[l, ch]],
                                      c_v.at[pl.ds(ch * CHUNK, CHUNK)],
                                      sem_c).wait()
            return carry
        lax.fori_loop(1, Lw, drain, 0)
        for dsc in rows:
            dsc.wait()

        def group(g, carry):
            r = g * LANES + iot

            def dot_step(d, acc):
                dv = jnp.full((LANES,), d, jnp.int32)
                return acc + (plsc.load_gather(u_v, [r, dv]) *
                              plsc.load_gather(c_v, [r, dv]))

            s_c = lax.fori_loop(0, D, dot_step, jnp.zeros((LANES,),
                                                          jnp.float32))
            cu_v[pl.ds(g * LANES, LANES)] = s_c
            return carry
        lax.fori_loop(0, ngr, group, 0)

        base = w * bpw
        pltpu.sync_copy(u_v, ustage_h.at[pl.ds(base, bpw)])
        pltpu.sync_copy(cu_v, cu_h.at[pl.ds(base, bpw)])

    @functools.partial(
        pl.kernel,
        out_type=(jax.ShapeDtypeStruct((B,), jnp.float32),
                  jax.ShapeDtypeStruct((B,), jnp.float32)),
        mesh=mesh,
        compiler_params=_PARAMS,
        scratch_types=[
            pltpu.VMEM((nch, CHUNK), jnp.int32),
            pltpu.VMEM((nch, CHUNK), jnp.int32),
            pltpu.VMEM((bpw, D), jnp.float32),
            pltpu.VMEM((bpw, D), jnp.float32),
            pltpu.VMEM((bpw, D), jnp.float32),
            pltpu.VMEM((bpw,), jnp.float32),
            pltpu.VMEM((bpw,), jnp.float32),
            pltpu.VMEM((bpw,), jnp.float32),
            pltpu.SemaphoreType.DMA,
        ],
    )
    def run_b(pidx_h, nidx_h, iw_h, ustage_h, cu_h, spos_h, sneg_h,
              pidx_v, nidx_v, u_v, ip_v, in_v, cu_v, sp_v, sn_v, sem_rows):
        w = lax.axis_index("s") * NC + lax.axis_index("c")
        iot = lax.iota(jnp.int32, LANES)
        base = w * bpw

        pltpu.sync_copy(pidx_h.at[w], pidx_v)
        pltpu.sync_copy(nidx_h.at[w], nidx_v)
        pltpu.sync_copy(ustage_h.at[pl.ds(base, bpw)], u_v)
        pltpu.sync_copy(cu_h.at[pl.ds(base, bpw)], cu_v)

        rows = []
        for ch in range(nch):
            sl = pl.ds(ch * CHUNK, CHUNK)
            rows.append(pltpu.async_copy(iw_h.at[pidx_v.at[ch]],
                                         ip_v.at[sl], sem_rows))
            rows.append(pltpu.async_copy(iw_h.at[nidx_v.at[ch]],
                                         in_v.at[sl], sem_rows))
        for dsc in rows:
            dsc.wait()

        inv_l = jnp.float32(1.0 / Lw)

        def group(g, carry):
            r = g * LANES + iot
            sl = pl.ds(g * LANES, LANES)

            def dot_step(d, acc):
                s_p, s_n = acc
                dv = jnp.full((LANES,), d, jnp.int32)
                uu = plsc.load_gather(u_v, [r, dv])
                s_p = s_p + uu * plsc.load_gather(ip_v, [r, dv])
                s_n = s_n + uu * plsc.load_gather(in_v, [r, dv])
                return (s_p, s_n)

            z = jnp.zeros((LANES,), jnp.float32)
            s_p, s_n = lax.fori_loop(0, D, dot_step, (z, z))
            sc = cu_v[sl] * inv_l
            sp_v[sl] = s_p + sc
            sn_v[sl] = s_n + sc
            return carry
        lax.fori_loop(0, ngr, group, 0)

        pltpu.sync_copy(sp_v, spos_h.at[pl.ds(base, bpw)])
        pltpu.sync_copy(sn_v, sneg_h.at[pl.ds(base, bpw)])

    ustage, cu = run_a(uidx, cidx, user_embed_w, text_embed_w)
    sp, sn = run_b(pidx, nidx, item_embed_w, ustage, cu)
    return sp, sn


# speedup vs baseline: 4.1197x; 1.0067x over previous
import functools

import jax
import jax.numpy as jnp
from jax import lax
from jax.experimental import pallas as pl
from jax.experimental.pallas import tpu as pltpu
from jax.experimental.pallas import tpu_sc as plsc

NC = 2
NS = 16
NW = NC * NS
LANES = 16
CHUNK = 128

_PARAMS = pltpu.CompilerParams(needs_layout_passes=False,
                               use_tc_tiling_on_sc=False)


def kernel(user, pos_item, neg_item, comment_tensor,
           user_embed_w, item_embed_w, text_embed_w):
    B = user.shape[0]
    Lw = comment_tensor.shape[1]
    D = user_embed_w.shape[1]
    bpw = B // NW
    nch = bpw // CHUNK
    ngr = bpw // LANES

    uidx = user.astype(jnp.int32).reshape(NW, nch, CHUNK)
    pidx = pos_item.astype(jnp.int32).reshape(NW, nch, CHUNK)
    nidx = neg_item.astype(jnp.int32).reshape(NW, nch, CHUNK)
    cidx = comment_tensor.T.astype(jnp.int32).reshape(Lw, NW, nch, CHUNK)

    mesh = plsc.VectorSubcoreMesh(core_axis_name="c", subcore_axis_name="s")

    @functools.partial(
        pl.kernel,
        out_type=(jax.ShapeDtypeStruct((B, D), jnp.float32),
                  jax.ShapeDtypeStruct((B,), jnp.float32)),
        mesh=mesh,
        compiler_params=_PARAMS,
        scratch_types=[
            pltpu.VMEM((nch, CHUNK), jnp.int32),
            pltpu.VMEM((Lw, nch, CHUNK), jnp.int32),
            pltpu.VMEM((bpw, D), jnp.float32),
            pltpu.VMEM((bpw, D), jnp.float32),
            pltpu.VMEM((bpw,), jnp.float32),
            pltpu.SemaphoreType.DMA,
            pltpu.SemaphoreType.DMA,
        ],
    )
    def run_a(uidx_h, cidx_h, uw_h, tw_h, ustage_h, cu_h,
              uidx_v, cidx_v, u_v, c_v, cu_v, sem_rows, sem_c):
        w = lax.axis_index("s") * NC + lax.axis_index("c")
        iot = lax.iota(jnp.int32, LANES)

        pltpu.sync_copy(uidx_h.at[w], uidx_v)
        pltpu.sync_copy(cidx_h.at[:, w], cidx_v)

        first = []
        for ch in range(nch):
            first.append(pltpu.async_copy(
                tw_h.at[cidx_v.at[0, ch]],
                c_v.at[pl.ds(ch * CHUNK, CHUNK)], sem_c))
        rows = []
        for ch in range(nch):
            rows.append(pltpu.async_copy(
                uw_h.at[uidx_v.at[ch]],
                u_v.at[pl.ds(ch * CHUNK, CHUNK)], sem_rows))
        for dsc in first:
            dsc.wait()

        def fire(l, carry):
            for ch in range(nch):
                pltpu.async_copy(tw_h.at[cidx_v.at[l, ch]],
                                 c_v.at[pl.ds(ch * CHUNK, CHUNK)],
                                 sem_c, add=True)
            return carry
        lax.fori_loop(1, Lw, fire, 0)

        def drain(l, carry):
            for ch in range(nch):
                pltpu.make_async_copy(tw_h.at[cidx_v.at[l, ch]],
                                      c_v.at[pl.ds(ch * CHUNK, CHUNK)],
                                      sem_c).wait()
            return carry
        lax.fori_loop(1, Lw, drain, 0)
        for dsc in rows:
            dsc.wait()

        def group(g, carry):
            r = g * LANES + iot

            def dot_step(d, acc):
                dv = jnp.full((LANES,), d, jnp.int32)
                return acc + (plsc.load_gather(u_v, [r, dv]) *
                              plsc.load_gather(c_v, [r, dv]))

            s_c = lax.fori_loop(0, D, dot_step, jnp.zeros((LANES,),
                                                          jnp.float32))
            cu_v[pl.ds(g * LANES, LANES)] = s_c
            return carry
        lax.fori_loop(0, ngr, group, 0)

        base = w * bpw
        pltpu.sync_copy(u_v, ustage_h.at[pl.ds(base, bpw)])
        pltpu.sync_copy(cu_v, cu_h.at[pl.ds(base, bpw)])

    @functools.partial(
        pl.kernel,
        out_type=(jax.ShapeDtypeStruct((B,), jnp.float32),
                  jax.ShapeDtypeStruct((B,), jnp.float32)),
        mesh=mesh,
        compiler_params=_PARAMS,
        scratch_types=[
            pltpu.VMEM((nch, CHUNK), jnp.int32),
            pltpu.VMEM((nch, CHUNK), jnp.int32),
            pltpu.VMEM((bpw, D), jnp.float32),
            pltpu.VMEM((bpw, D), jnp.float32),
            pltpu.VMEM((bpw, D), jnp.float32),
            pltpu.VMEM((bpw,), jnp.float32),
            pltpu.VMEM((bpw,), jnp.float32),
            pltpu.VMEM((bpw,), jnp.float32),
            pltpu.SemaphoreType.DMA,
        ],
    )
    def run_b(pidx_h, nidx_h, iw_h, ustage_h, cu_h, spos_h, sneg_h,
              pidx_v, nidx_v, u_v, ip_v, in_v, cu_v, sp_v, sn_v, sem_rows):
        w = lax.axis_index("s") * NC + lax.axis_index("c")
        iot = lax.iota(jnp.int32, LANES)
        base = w * bpw

        pltpu.sync_copy(pidx_h.at[w], pidx_v)
        pltpu.sync_copy(nidx_h.at[w], nidx_v)

        rows = []
        for ch in range(nch):
            sl = pl.ds(ch * CHUNK, CHUNK)
            rows.append(pltpu.async_copy(iw_h.at[pidx_v.at[ch]],
                                         ip_v.at[sl], sem_rows))
            rows.append(pltpu.async_copy(iw_h.at[nidx_v.at[ch]],
                                         in_v.at[sl], sem_rows))
        pltpu.sync_copy(ustage_h.at[pl.ds(base, bpw)], u_v)
        pltpu.sync_copy(cu_h.at[pl.ds(base, bpw)], cu_v)
        for dsc in rows:
            dsc.wait()

        inv_l = jnp.float32(1.0 / Lw)

        def group(g, carry):
            r = g * LANES + iot
            sl = pl.ds(g * LANES, LANES)

            z = jnp.zeros((LANES,), jnp.float32)
            s_p = z
            s_n = z
            for d in range(D):
                dv = jnp.full((LANES,), d, jnp.int32)
                uu = plsc.load_gather(u_v, [r, dv])
                s_p = s_p + uu * plsc.load_gather(ip_v, [r, dv])
                s_n = s_n + uu * plsc.load_gather(in_v, [r, dv])
            sc = cu_v[sl] * inv_l
            sp_v[sl] = s_p + sc
            sn_v[sl] = s_n + sc
            return carry
        lax.fori_loop(0, ngr, group, 0)

        pltpu.sync_copy(sp_v, spos_h.at[pl.ds(base, bpw)])
        pltpu.sync_copy(sn_v, sneg_h.at[pl.ds(base, bpw)])

    ustage, cu = run_a(uidx, cidx, user_embed_w, text_embed_w)
    sp, sn = run_b(pidx, nidx, item_embed_w, ustage, cu)
    return sp, sn
